# baseline (speedup 1.0000x reference)
import math

import jax
import jax.numpy as jnp
from jax.experimental import pallas as pl
from jax.experimental.pallas import tpu as pltpu

_PAD = 0
_CONF = 0.9
_N = 2048
_V = 32000
_R = 32
_GRID = _N // _R
_SLABS = _V // 128

_L1 = _CONF
_L0 = (1.0 - _CONF) / (_V - 2)
_C = _L1 * math.log(_L1) + (_V - 1) * _L0 * math.log(_L0)


def _body(yts_ref, m_ref, yp_ref, loss_ref, npad_ref):
    i = pl.program_id(0)
    lane = jax.lax.broadcasted_iota(jnp.int32, (_R, 128), 1)
    d = yts_ref[...] - lane

    part_s = yp_ref[:, 0:128]
    part_g = jnp.where(d == 0, part_s, 0.0)
    for c in range(1, _SLABS):
        slab = yp_ref[:, c * 128:(c + 1) * 128]
        part_s = part_s + slab
        part_g = part_g + jnp.where(d == c * 128, slab, 0.0)

    m = m_ref[...]
    contrib = (
        jnp.sum(m) * _C
        - _L0 * jnp.sum(part_s * m)
        - (_L1 - _L0) * jnp.sum(part_g)
    )
    npad_part = jnp.sum(m).astype(jnp.int32)

    @pl.when(i == 0)
    def _():
        loss_ref[0, 0] = 0.0
        npad_ref[0, 0] = 0

    loss_ref[0, 0] += contrib
    npad_ref[0, 0] += npad_part


def kernel(y_pred, y_true):
    yp = y_pred.reshape(_N, _V)
    yt = y_true.reshape(_N, 1)
    nonpad = yt != _PAD
    yts = jnp.where(nonpad, yt, -1)
    mrow = nonpad.astype(jnp.float32)

    loss, npad = pl.pallas_call(
        _body,
        grid=(_GRID,),
        in_specs=[
            pl.BlockSpec((_R, 1), lambda i: (i, 0)),
            pl.BlockSpec((_R, 1), lambda i: (i, 0)),
            pl.BlockSpec((_R, _V), lambda i: (i, 0)),
        ],
        out_specs=[
            pl.BlockSpec(memory_space=pltpu.SMEM),
            pl.BlockSpec(memory_space=pltpu.SMEM),
        ],
        out_shape=[
            jax.ShapeDtypeStruct((1, 1), jnp.float32),
            jax.ShapeDtypeStruct((1, 1), jnp.int32),
        ],
    )(yts, mrow, yp)
    return (loss[0, 0], npad[0, 0])


# speedup vs baseline: 1.2230x; 1.2230x over previous
import math

import jax
import jax.numpy as jnp
from jax.experimental import pallas as pl
from jax.experimental.pallas import tpu as pltpu

_PAD = 0
_CONF = 0.9
_N = 2048
_V = 32000
_R = 64
_GRID = _N // _R
_SLABS = _V // 128

_L1 = _CONF
_L0 = (1.0 - _CONF) / (_V - 2)
_C = _L1 * math.log(_L1) + (_V - 1) * _L0 * math.log(_L0)


def _body(yts_ref, m_ref, yp_ref, loss_ref, npad_ref):
    i = pl.program_id(0)
    lane = jax.lax.broadcasted_iota(jnp.int32, (_R, 128), 1)
    d = yts_ref[...] - lane

    part_s = yp_ref[:, 0:128]
    part_g = jnp.where(d == 0, part_s, 0.0)
    for c in range(1, _SLABS):
        slab = yp_ref[:, c * 128:(c + 1) * 128]
        part_s = part_s + slab
        part_g = part_g + jnp.where(d == c * 128, slab, 0.0)

    m = m_ref[...]
    contrib = (
        jnp.sum(m) * _C
        - _L0 * jnp.sum(part_s * m)
        - (_L1 - _L0) * jnp.sum(part_g)
    )
    npad_part = jnp.sum(m).astype(jnp.int32)

    @pl.when(i == 0)
    def _():
        loss_ref[0, 0] = 0.0
        npad_ref[0, 0] = 0

    loss_ref[0, 0] += contrib
    npad_ref[0, 0] += npad_part


def kernel(y_pred, y_true):
    yp = y_pred.reshape(_N, _V)
    yt = y_true.reshape(_N, 1)
    nonpad = yt != _PAD
    yts = jnp.where(nonpad, yt, -1)
    mrow = nonpad.astype(jnp.float32)

    loss, npad = pl.pallas_call(
        _body,
        grid=(_GRID,),
        in_specs=[
            pl.BlockSpec((_R, 1), lambda i: (i, 0)),
            pl.BlockSpec((_R, 1), lambda i: (i, 0)),
            pl.BlockSpec((_R, _V), lambda i: (i, 0)),
        ],
        out_specs=[
            pl.BlockSpec(memory_space=pltpu.SMEM),
            pl.BlockSpec(memory_space=pltpu.SMEM),
        ],
        out_shape=[
            jax.ShapeDtypeStruct((1, 1), jnp.float32),
            jax.ShapeDtypeStruct((1, 1), jnp.int32),
        ],
    )(yts, mrow, yp)
    return (loss[0, 0], npad[0, 0])


# speedup vs baseline: 1.2461x; 1.0189x over previous
import math

import jax
import jax.numpy as jnp
from jax.experimental import pallas as pl
from jax.experimental.pallas import tpu as pltpu

_PAD = 0
_CONF = 0.9
_N = 2048
_V = 32000
_R = 64
_GRID = _N // _R
_SLABS = _V // 128

_L1 = _CONF
_L0 = (1.0 - _CONF) / (_V - 2)
_C = _L1 * math.log(_L1) + (_V - 1) * _L0 * math.log(_L0)


def _body(yts_ref, yp_ref, loss_ref, npad_ref):
    i = pl.program_id(0)
    yts = yts_ref[...]
    lane = jax.lax.broadcasted_iota(jnp.int32, (_R, 128), 1)
    d = yts - lane

    part_s = yp_ref[:, 0:128]
    part_g = jnp.where(d == 0, part_s, 0.0)
    for c in range(1, _SLABS):
        slab = yp_ref[:, c * 128:(c + 1) * 128]
        part_s = part_s + slab
        part_g = part_g + jnp.where(d == c * 128, slab, 0.0)

    m = (yts >= 0).astype(jnp.float32)
    contrib = (
        jnp.sum(m) * _C
        - _L0 * jnp.sum(part_s * m)
        - (_L1 - _L0) * jnp.sum(part_g)
    )
    npad_part = jnp.sum(m).astype(jnp.int32)

    @pl.when(i == 0)
    def _():
        loss_ref[0, 0] = 0.0
        npad_ref[0, 0] = 0

    loss_ref[0, 0] += contrib
    npad_ref[0, 0] += npad_part


def kernel(y_pred, y_true):
    yp = y_pred.reshape(_N, _V)
    yt = y_true.reshape(_N, 1)
    yts = jnp.where(yt != _PAD, yt, -1)

    loss, npad = pl.pallas_call(
        _body,
        grid=(_GRID,),
        in_specs=[
            pl.BlockSpec((_R, 1), lambda i: (i, 0)),
            pl.BlockSpec((_R, _V), lambda i: (i, 0)),
        ],
        out_specs=[
            pl.BlockSpec(memory_space=pltpu.SMEM),
            pl.BlockSpec(memory_space=pltpu.SMEM),
        ],
        out_shape=[
            jax.ShapeDtypeStruct((1, 1), jnp.float32),
            jax.ShapeDtypeStruct((1, 1), jnp.int32),
        ],
    )(yts, yp)
    return (loss[0, 0], npad[0, 0])
